# Initial kernel scaffold; baseline (speedup 1.0000x reference)
import jax
import jax.numpy as jnp
from jax.experimental import pallas as pl


def kernel(observations, actions, atten_masks, W_op, b_op, ln1_g, ln1_b, Wk, bk, Wv, bv, Wq, bq, ln2_g, ln2_b, Wp, bp, ln3_g, ln3_b):
    raise NotImplementedError("write your pallas kernel here")


# speedup vs baseline: 3.7353x; 3.7353x over previous
import functools

import jax
import jax.numpy as jnp
from jax.experimental import pallas as pl

B, L, D, H = 2, 256, 512, 8
DH = D // H
_BL = B * L


def _layernorm(x, g, b, eps=1e-5):
    m = jnp.mean(x, axis=-1, keepdims=True)
    v = jnp.mean((x - m) ** 2, axis=-1, keepdims=True)
    return (x - m) * jax.lax.rsqrt(v + eps) * g + b


def _gelu(x):
    return x * 0.5 * (1.0 + jax.lax.erf(x * (2.0 ** -0.5)))


def _dot(a, b):
    return jnp.dot(a, b, preferred_element_type=jnp.float32)


def _dot_t(a, b):
    return jax.lax.dot_general(
        a, b, (((1,), (1,)), ((), ())), preferred_element_type=jnp.float32)


def _fused_kernel(obs_ref, act_ref, mask_ref,
                  wq_ref, wko_ref, wka_ref, wvo_ref, wva_ref,
                  wop_ref, wp_ref, vecs_ref, out_ref):
    obs = obs_ref[...]
    act = act_ref[...]

    bq = vecs_ref[0, :D][None, :]
    bk = vecs_ref[1, :D][None, :]
    bv = vecs_ref[2, :D][None, :]
    b_op = vecs_ref[3, :D][None, :]
    ln1_g = vecs_ref[4, :D][None, :]
    ln1_b = vecs_ref[5, :D][None, :]
    bp = vecs_ref[6, :D][None, :]
    ln3_g = vecs_ref[7, :D][None, :]
    ln3_b = vecs_ref[8, :D][None, :]
    ln2_g = vecs_ref[9, :][None, :]
    ln2_b = vecs_ref[10, :][None, :]

    q = _dot(obs, wq_ref[...]) + bq
    k = _dot(obs, wko_ref[...]) + _dot(act, wka_ref[...]) + bk
    v = _dot(obs, wvo_ref[...]) + _dot(act, wva_ref[...]) + bv

    qs = jnp.concatenate([q[_BL - 1:, :], q[:_BL - 1, :]], axis=0)

    row = jax.lax.broadcasted_iota(jnp.int32, (L, L), 0)
    col = jax.lax.broadcasted_iota(jnp.int32, (L, L), 1)
    not_subdiag = col != (row + (L - 1)) % L

    batch_rows = []
    for b in range(B):
        maskb = (mask_ref[b] != 0) & not_subdiag
        qb = qs[b * L:(b + 1) * L, :]
        kb = k[b * L:(b + 1) * L, :]
        vb = v[b * L:(b + 1) * L, :]
        head_out = []
        for h in range(H):
            qh = qb[:, h * DH:(h + 1) * DH]
            kh = kb[:, h * DH:(h + 1) * DH]
            vh = vb[:, h * DH:(h + 1) * DH]
            s = _dot_t(qh, kh)
            wm = jnp.where(maskb, s, -jnp.inf)
            wmax = jnp.max(wm, axis=1, keepdims=True)
            wmax = jnp.where(wmax > -jnp.inf, wmax, 0.0)
            ex = jnp.where(maskb, jnp.exp(s - wmax), 0.0)
            denom = jnp.sum(ex, axis=1, keepdims=True)
            attn = ex / (denom + 1e-16)
            head_out.append(_dot(attn, vh))
        batch_rows.append(jnp.concatenate(head_out, axis=1))
    y = jnp.concatenate(batch_rows, axis=0)

    y = jnp.concatenate([y[1:, :], y[:1, :]], axis=0)

    obs_branch = _layernorm(_gelu(_dot(obs, wop_ref[...]) + b_op), ln1_g, ln1_b)
    cat = jnp.concatenate([y, obs_branch], axis=1)
    cat = _layernorm(cat, ln2_g, ln2_b)
    z = _gelu(_dot(cat, wp_ref[...]) + bp)
    out_ref[...] = _layernorm(z, ln3_g, ln3_b)


@jax.jit
def kernel(observations, actions, atten_masks, W_op, b_op, ln1_g, ln1_b,
           Wk, bk, Wv, bv, Wq, bq, ln2_g, ln2_b, Wp, bp, ln3_g, ln3_b):
    obs = observations.reshape(_BL, D)
    act = actions.reshape(_BL, D)

    pad = jnp.zeros((D,), jnp.float32)
    vecs = jnp.stack([
        jnp.concatenate([bq, pad]), jnp.concatenate([bk, pad]),
        jnp.concatenate([bv, pad]), jnp.concatenate([b_op, pad]),
        jnp.concatenate([ln1_g, pad]), jnp.concatenate([ln1_b, pad]),
        jnp.concatenate([bp, pad]), jnp.concatenate([ln3_g, pad]),
        jnp.concatenate([ln3_b, pad]), ln2_g, ln2_b,
        pad2 := jnp.zeros((2 * D,), jnp.float32), pad2, pad2, pad2, pad2,
    ])

    out = pl.pallas_call(
        _fused_kernel,
        out_shape=jax.ShapeDtypeStruct((_BL, D), jnp.float32),
    )(obs, act, atten_masks,
      Wq.T, Wk[:, :D].T, Wk[:, D:].T, Wv[:, :D].T, Wv[:, D:].T,
      W_op.T, Wp.T, vecs)
    return out.reshape(B, L, D)


# speedup vs baseline: 5.2036x; 1.3931x over previous
import jax
import jax.numpy as jnp
from jax.experimental import pallas as pl

B, L, D, H = 2, 256, 512, 8
DH = D // H
_BL = B * L


def _layernorm(x, g, b, eps=1e-5):
    m = jnp.mean(x, axis=-1, keepdims=True)
    v = jnp.mean((x - m) ** 2, axis=-1, keepdims=True)
    return (x - m) * jax.lax.rsqrt(v + eps) * g + b


def _gelu(x):
    return x * 0.5 * (1.0 + jax.lax.erf(x * (2.0 ** -0.5)))


def _dot(a, b):
    return jnp.dot(a, b, preferred_element_type=jnp.float32)


def _dot_t(a, b):
    return jax.lax.dot_general(
        a, b, (((1,), (1,)), ((), ())), preferred_element_type=jnp.float32)


def _fused_kernel(obs_ref, act_ref, mask_ref,
                  wq_ref, wko_ref, wka_ref, wvo_ref, wva_ref,
                  wop_ref, wp_ref, vecs_ref, out_ref):
    obs = obs_ref[...]
    act = act_ref[...]

    bq = vecs_ref[0, :D][None, :]
    bk = vecs_ref[1, :D][None, :]
    bv = vecs_ref[2, :D][None, :]
    b_op = vecs_ref[3, :D][None, :]
    ln1_g = vecs_ref[4, :D][None, :]
    ln1_b = vecs_ref[5, :D][None, :]
    bp = vecs_ref[6, :D][None, :]
    ln3_g = vecs_ref[7, :D][None, :]
    ln3_b = vecs_ref[8, :D][None, :]
    ln2_g = vecs_ref[9, :][None, :]
    ln2_b = vecs_ref[10, :][None, :]

    q = _dot(obs, wq_ref[...]) + bq
    k = _dot(obs, wko_ref[...]) + _dot(act, wka_ref[...]) + bk
    v = _dot(obs, wvo_ref[...]) + _dot(act, wva_ref[...]) + bv

    qs = jnp.concatenate([q[_BL - 1:, :], q[:_BL - 1, :]], axis=0)
    qs = qs.astype(jnp.bfloat16)
    k = k.astype(jnp.bfloat16)
    v = v.astype(jnp.bfloat16)

    row = jax.lax.broadcasted_iota(jnp.int32, (L, L), 0)
    col = jax.lax.broadcasted_iota(jnp.int32, (L, L), 1)
    not_subdiag = col != (row + (L - 1)) % L

    batch_rows = []
    for b in range(B):
        valid = (mask_ref[b] != 0) & not_subdiag
        bias = jnp.where(valid, 0.0, -1e30)
        qb = qs[b * L:(b + 1) * L, :]
        kb = k[b * L:(b + 1) * L, :]
        vb = v[b * L:(b + 1) * L, :]
        head_out = []
        for h in range(H):
            qh = qb[:, h * DH:(h + 1) * DH]
            kh = kb[:, h * DH:(h + 1) * DH]
            vh = vb[:, h * DH:(h + 1) * DH]
            s = _dot_t(qh, kh) + bias
            wmax = jnp.maximum(jnp.max(s, axis=1, keepdims=True), -1e25)
            ex = jnp.exp(s - wmax)
            denom = jnp.sum(ex, axis=1, keepdims=True)
            recip = 1.0 / (denom + 1e-16)
            yh = _dot(ex.astype(jnp.bfloat16), vh) * recip
            head_out.append(yh)
        batch_rows.append(jnp.concatenate(head_out, axis=1))
    y = jnp.concatenate(batch_rows, axis=0)

    y = jnp.concatenate([y[1:, :], y[:1, :]], axis=0)

    obs_branch = _layernorm(_gelu(_dot(obs, wop_ref[...]) + b_op), ln1_g, ln1_b)
    cat = jnp.concatenate([y, obs_branch], axis=1)
    cat = _layernorm(cat, ln2_g, ln2_b).astype(jnp.bfloat16)
    z = _gelu(_dot(cat, wp_ref[...]) + bp)
    out_ref[...] = _layernorm(z, ln3_g, ln3_b)


@jax.jit
def kernel(observations, actions, atten_masks, W_op, b_op, ln1_g, ln1_b,
           Wk, bk, Wv, bv, Wq, bq, ln2_g, ln2_b, Wp, bp, ln3_g, ln3_b):
    bf = jnp.bfloat16
    obs = observations.reshape(_BL, D).astype(bf)
    act = actions.reshape(_BL, D).astype(bf)

    pad = jnp.zeros((D,), jnp.float32)
    vecs = jnp.stack([
        jnp.concatenate([bq, pad]), jnp.concatenate([bk, pad]),
        jnp.concatenate([bv, pad]), jnp.concatenate([b_op, pad]),
        jnp.concatenate([ln1_g, pad]), jnp.concatenate([ln1_b, pad]),
        jnp.concatenate([bp, pad]), jnp.concatenate([ln3_g, pad]),
        jnp.concatenate([ln3_b, pad]), ln2_g, ln2_b,
        pad2 := jnp.zeros((2 * D,), jnp.float32), pad2, pad2, pad2, pad2,
    ])

    out = pl.pallas_call(
        _fused_kernel,
        out_shape=jax.ShapeDtypeStruct((_BL, D), jnp.float32),
    )(obs, act, atten_masks,
      Wq.T.astype(bf), Wk[:, :D].T.astype(bf), Wk[:, D:].T.astype(bf),
      Wv[:, :D].T.astype(bf), Wv[:, D:].T.astype(bf),
      W_op.T.astype(bf), Wp.T.astype(bf), vecs)
    return out.reshape(B, L, D)


# speedup vs baseline: 5.3744x; 1.0328x over previous
import jax
import jax.numpy as jnp
from jax.experimental import pallas as pl
from jax.experimental.pallas import tpu as pltpu

B, L, D, H = 2, 256, 512, 8
DH = D // H
_BL = B * L


def _layernorm(x, g, b, eps=1e-5):
    m = jnp.mean(x, axis=-1, keepdims=True)
    v = jnp.mean((x - m) ** 2, axis=-1, keepdims=True)
    return (x - m) * jax.lax.rsqrt(v + eps) * g + b


def _gelu(x):
    return x * 0.5 * (1.0 + jax.lax.erf(x * (2.0 ** -0.5)))


def _dot(a, b):
    return jnp.dot(a, b, preferred_element_type=jnp.float32)


def _dot_t(a, b):
    return jax.lax.dot_general(
        a, b, (((1,), (1,)), ((), ())), preferred_element_type=jnp.float32)


def _fused_kernel(obs_hbm, act_hbm, mask_hbm,
                  wq_hbm, wko_hbm, wka_hbm, wvo_hbm, wva_hbm,
                  wop_hbm, wp_hbm, vecs_hbm, out_ref,
                  obs_v, act_v, mask_v, wq_v, wko_v, wka_v, wvo_v, wva_v,
                  wop_v, wp_v, vecs_v, sems):
    srcs = (obs_hbm, wq_hbm, vecs_hbm, act_hbm, wko_hbm, wka_hbm, wvo_hbm,
            wva_hbm, mask_hbm, wop_hbm, wp_hbm)
    dsts = (obs_v, wq_v, vecs_v, act_v, wko_v, wka_v, wvo_v,
            wva_v, mask_v, wop_v, wp_v)
    copies = [pltpu.make_async_copy(s, d, sems.at[i])
              for i, (s, d) in enumerate(zip(srcs, dsts))]
    for c in copies:
        c.start()
    (c_obs, c_wq, c_vecs, c_act, c_wko, c_wka, c_wvo, c_wva, c_mask,
     c_wop, c_wp) = copies

    c_vecs.wait()
    bq = vecs_v[0, :D][None, :]
    bk = vecs_v[1, :D][None, :]
    bv = vecs_v[2, :D][None, :]
    b_op = vecs_v[3, :D][None, :]
    ln1_g = vecs_v[4, :D][None, :]
    ln1_b = vecs_v[5, :D][None, :]
    bp = vecs_v[6, :D][None, :]
    ln3_g = vecs_v[7, :D][None, :]
    ln3_b = vecs_v[8, :D][None, :]
    ln2_g = vecs_v[9, :][None, :]
    ln2_b = vecs_v[10, :][None, :]

    c_obs.wait()
    c_wq.wait()
    obs = obs_v[...]
    q = _dot(obs, wq_v[...]) + bq
    c_act.wait()
    act = act_v[...]
    c_wko.wait()
    c_wka.wait()
    k = _dot(obs, wko_v[...]) + _dot(act, wka_v[...]) + bk
    c_wvo.wait()
    c_wva.wait()
    v = _dot(obs, wvo_v[...]) + _dot(act, wva_v[...]) + bv

    qs = jnp.concatenate([q[_BL - 1:, :], q[:_BL - 1, :]], axis=0)
    qs = qs.astype(jnp.bfloat16)
    k = k.astype(jnp.bfloat16)
    v = v.astype(jnp.bfloat16)

    row = jax.lax.broadcasted_iota(jnp.int32, (L, L), 0)
    col = jax.lax.broadcasted_iota(jnp.int32, (L, L), 1)
    not_subdiag = col != (row + (L - 1)) % L

    c_mask.wait()
    batch_rows = []
    for b in range(B):
        valid = (mask_v[b] != 0) & not_subdiag
        bias = jnp.where(valid, 0.0, -1e30)
        qb = qs[b * L:(b + 1) * L, :]
        kb = k[b * L:(b + 1) * L, :]
        vb = v[b * L:(b + 1) * L, :]
        head_out = []
        for h in range(H):
            qh = qb[:, h * DH:(h + 1) * DH]
            kh = kb[:, h * DH:(h + 1) * DH]
            vh = vb[:, h * DH:(h + 1) * DH]
            s = _dot_t(qh, kh) + bias
            wmax = jnp.maximum(jnp.max(s, axis=1, keepdims=True), -1e25)
            ex = jnp.exp(s - wmax)
            denom = jnp.sum(ex, axis=1, keepdims=True)
            recip = 1.0 / (denom + 1e-16)
            yh = _dot(ex.astype(jnp.bfloat16), vh) * recip
            head_out.append(yh)
        batch_rows.append(jnp.concatenate(head_out, axis=1))
    y = jnp.concatenate(batch_rows, axis=0)

    y = jnp.concatenate([y[1:, :], y[:1, :]], axis=0)

    c_wop.wait()
    obs_branch = _layernorm(_gelu(_dot(obs, wop_v[...]) + b_op), ln1_g, ln1_b)
    cat = jnp.concatenate([y, obs_branch], axis=1)
    cat = _layernorm(cat, ln2_g, ln2_b).astype(jnp.bfloat16)
    c_wp.wait()
    z = _gelu(_dot(cat, wp_v[...]) + bp)
    out_ref[...] = _layernorm(z, ln3_g, ln3_b)


@jax.jit
def kernel(observations, actions, atten_masks, W_op, b_op, ln1_g, ln1_b,
           Wk, bk, Wv, bv, Wq, bq, ln2_g, ln2_b, Wp, bp, ln3_g, ln3_b):
    bf = jnp.bfloat16
    obs = observations.reshape(_BL, D).astype(bf)
    act = actions.reshape(_BL, D).astype(bf)

    pad = jnp.zeros((D,), jnp.float32)
    vecs = jnp.stack([
        jnp.concatenate([bq, pad]), jnp.concatenate([bk, pad]),
        jnp.concatenate([bv, pad]), jnp.concatenate([b_op, pad]),
        jnp.concatenate([ln1_g, pad]), jnp.concatenate([ln1_b, pad]),
        jnp.concatenate([bp, pad]), jnp.concatenate([ln3_g, pad]),
        jnp.concatenate([ln3_b, pad]), ln2_g, ln2_b,
        pad2 := jnp.zeros((2 * D,), jnp.float32), pad2, pad2, pad2, pad2,
    ])

    hbm = pl.BlockSpec(memory_space=pltpu.MemorySpace.HBM)
    out = pl.pallas_call(
        _fused_kernel,
        out_shape=jax.ShapeDtypeStruct((_BL, D), jnp.float32),
        in_specs=[hbm] * 11,
        scratch_shapes=[
            pltpu.VMEM((_BL, D), bf),
            pltpu.VMEM((_BL, D), bf),
            pltpu.VMEM((B, L, L), jnp.int32),
            pltpu.VMEM((D, D), bf),
            pltpu.VMEM((D, D), bf),
            pltpu.VMEM((D, D), bf),
            pltpu.VMEM((D, D), bf),
            pltpu.VMEM((D, D), bf),
            pltpu.VMEM((D, D), bf),
            pltpu.VMEM((2 * D, D), bf),
            pltpu.VMEM((16, 2 * D), jnp.float32),
            pltpu.SemaphoreType.DMA((11,)),
        ],
    )(obs, act, atten_masks,
      Wq.T.astype(bf), Wk[:, :D].T.astype(bf), Wk[:, D:].T.astype(bf),
      Wv[:, :D].T.astype(bf), Wv[:, D:].T.astype(bf),
      W_op.T.astype(bf), Wp.T.astype(bf), vecs)
    return out.reshape(B, L, D)


# speedup vs baseline: 9.3084x; 1.7320x over previous
import jax
import jax.numpy as jnp
from jax.experimental import pallas as pl
from jax.experimental.pallas import tpu as pltpu

B, L, D, H = 2, 256, 512, 8
DH = D // H
_BL = B * L
_BF = jnp.bfloat16


def _layernorm(x, g, b, eps=1e-5):
    m = jnp.mean(x, axis=-1, keepdims=True)
    v = jnp.mean((x - m) ** 2, axis=-1, keepdims=True)
    return (x - m) * jax.lax.rsqrt(v + eps) * g + b


def _gelu(x):
    return x * 0.5 * (1.0 + jax.lax.erf(x * (2.0 ** -0.5)))


def _dot(a, b):
    return jnp.dot(a, b, preferred_element_type=jnp.float32)


def _dot_t(a, b):
    return jax.lax.dot_general(
        a, b, (((1,), (1,)), ((), ())), preferred_element_type=jnp.float32)


def _fused_kernel(obs_hbm, act_hbm, mask_hbm, wq_hbm, wk_hbm, wv_hbm,
                  wop_hbm, wp_hbm,
                  bq_ref, bk_ref, bv_ref, b_op_ref, ln1_g_ref, ln1_b_ref,
                  bp_ref, ln3_g_ref, ln3_b_ref, ln2_g_ref, ln2_b_ref,
                  out_ref,
                  obs_v, act_v, mask_v, wq_v, wk_v, wv_v, wop_v, wp_v, sems):
    srcs = (obs_hbm, wq_hbm, act_hbm, wk_hbm, wv_hbm, mask_hbm, wop_hbm,
            wp_hbm)
    dsts = (obs_v, wq_v, act_v, wk_v, wv_v, mask_v, wop_v, wp_v)
    copies = [pltpu.make_async_copy(s, d, sems.at[i])
              for i, (s, d) in enumerate(zip(srcs, dsts))]
    for c in copies:
        c.start()
    c_obs, c_wq, c_act, c_wk, c_wv, c_mask, c_wop, c_wp = copies

    c_obs.wait()
    obs = obs_v[...].astype(_BF)
    c_wq.wait()
    q = _dot_t(obs, wq_v[...].astype(_BF)) + bq_ref[...]
    c_act.wait()
    act = act_v[...].astype(_BF)
    c_wk.wait()
    wk = wk_v[...].astype(_BF)
    k = _dot_t(obs, wk[:, :D]) + _dot_t(act, wk[:, D:]) + bk_ref[...]
    c_wv.wait()
    wv = wv_v[...].astype(_BF)
    v = _dot_t(obs, wv[:, :D]) + _dot_t(act, wv[:, D:]) + bv_ref[...]

    qs = jnp.concatenate([q[_BL - 1:, :], q[:_BL - 1, :]], axis=0)
    qs = qs.astype(_BF)
    k = k.astype(_BF)
    v = v.astype(_BF)

    row = jax.lax.broadcasted_iota(jnp.int32, (L, L), 0)
    col = jax.lax.broadcasted_iota(jnp.int32, (L, L), 1)
    not_subdiag = col != (row + (L - 1)) % L

    c_mask.wait()
    batch_rows = []
    for b in range(B):
        valid = (mask_v[b] != 0) & not_subdiag
        bias = jnp.where(valid, 0.0, -1e30)
        qb = qs[b * L:(b + 1) * L, :]
        kb = k[b * L:(b + 1) * L, :]
        vb = v[b * L:(b + 1) * L, :]
        head_out = []
        for h in range(H):
            qh = qb[:, h * DH:(h + 1) * DH]
            kh = kb[:, h * DH:(h + 1) * DH]
            vh = vb[:, h * DH:(h + 1) * DH]
            s = _dot_t(qh, kh) + bias
            wmax = jnp.maximum(jnp.max(s, axis=1, keepdims=True), -1e25)
            ex = jnp.exp(s - wmax)
            denom = jnp.sum(ex, axis=1, keepdims=True)
            recip = 1.0 / (denom + 1e-16)
            yh = _dot(ex.astype(_BF), vh) * recip
            head_out.append(yh)
        batch_rows.append(jnp.concatenate(head_out, axis=1))
    y = jnp.concatenate(batch_rows, axis=0)

    y = jnp.concatenate([y[1:, :], y[:1, :]], axis=0)

    c_wop.wait()
    obs_branch = _layernorm(
        _gelu(_dot_t(obs, wop_v[...].astype(_BF)) + b_op_ref[...]),
        ln1_g_ref[...], ln1_b_ref[...])
    cat = jnp.concatenate([y, obs_branch], axis=1)
    cat = _layernorm(cat, ln2_g_ref[...], ln2_b_ref[...]).astype(_BF)
    c_wp.wait()
    z = _gelu(_dot_t(cat, wp_v[...].astype(_BF)) + bp_ref[...])
    out_ref[...] = _layernorm(z, ln3_g_ref[...], ln3_b_ref[...])


@jax.jit
def kernel(observations, actions, atten_masks, W_op, b_op, ln1_g, ln1_b,
           Wk, bk, Wv, bv, Wq, bq, ln2_g, ln2_b, Wp, bp, ln3_g, ln3_b):
    hbm = pl.BlockSpec(memory_space=pltpu.MemorySpace.HBM)
    vmem = pl.BlockSpec(memory_space=pltpu.MemorySpace.VMEM)
    f32 = jnp.float32
    out = pl.pallas_call(
        _fused_kernel,
        out_shape=jax.ShapeDtypeStruct((_BL, D), f32),
        in_specs=[hbm] * 8 + [vmem] * 11,
        scratch_shapes=[
            pltpu.VMEM((_BL, D), f32),
            pltpu.VMEM((_BL, D), f32),
            pltpu.VMEM((B, L, L), jnp.int32),
            pltpu.VMEM((D, D), f32),
            pltpu.VMEM((D, 2 * D), f32),
            pltpu.VMEM((D, 2 * D), f32),
            pltpu.VMEM((D, D), f32),
            pltpu.VMEM((D, 2 * D), f32),
            pltpu.SemaphoreType.DMA((8,)),
        ],
    )(observations.reshape(_BL, D), actions.reshape(_BL, D), atten_masks,
      Wq, Wk, Wv, W_op, Wp,
      bq.reshape(1, D), bk.reshape(1, D), bv.reshape(1, D),
      b_op.reshape(1, D), ln1_g.reshape(1, D), ln1_b.reshape(1, D),
      bp.reshape(1, D), ln3_g.reshape(1, D), ln3_b.reshape(1, D),
      ln2_g.reshape(1, 2 * D), ln2_b.reshape(1, 2 * D))
    return out.reshape(B, L, D)
